# baseline (speedup 1.0000x reference)
import jax
import jax.numpy as jnp
from jax import lax
from jax.experimental import pallas as pl
from jax.experimental.pallas import tpu as pltpu
from jax.experimental.pallas import tpu_sc as plsc

_B, _L, _D, _N = 8, 2048, 16, 32
_NC, _NS = 2, 16
_NW = _NC * _NS
_ROWS = _B * _D * _N * _L // 128
_SROWS = 256
_UNITS = _ROWS // _SROWS
_UPW = _UNITS // _NW


def _scan_body(x_hbm, out_hbm, buf0, buf1, is0, is1, os0, os1):
    wid = lax.axis_index("s") * _NC + lax.axis_index("c")
    u0 = wid * _UPW
    bufs = (buf0, buf1)
    isems, osems = (is0, is1), (os0, os1)

    def in_cp(k):
        return pltpu.async_copy(
            x_hbm.at[pl.ds((u0 + k) * _SROWS, _SROWS), :],
            bufs[k % 2], isems[k % 2])

    def out_cp(k):
        return pltpu.async_copy(
            bufs[k % 2],
            out_hbm.at[pl.ds((u0 + k) * _SROWS, _SROWS), :],
            osems[k % 2])

    lane = lax.iota(jnp.int32, 16)
    rbase = (lane // 8) * 128 + (lane % 8)

    in_h = [in_cp(0), None]
    out_h = [None, None]
    for k in range(_UPW):
        s = k % 2
        if k + 1 < _UPW:
            if out_h[(k + 1) % 2] is not None:
                out_h[(k + 1) % 2].wait()
                out_h[(k + 1) % 2] = None
            in_h[(k + 1) % 2] = in_cp(k + 1)
        in_h[s].wait()
        ib = bufs[s]

        def block(tc, acc0):
            ridx = rbase + tc * 8

            @plsc.parallel_loop(0, 128, unroll=8, carry=acc0)
            def body(u, acc):
                cvec = jnp.full((16,), u, jnp.int32)
                v = plsc.load_gather(ib, [ridx, cvec])
                acc = acc + v
                plsc.store_scatter(ib, [ridx, cvec], acc)
                return acc

            return body

        lax.fori_loop(0, 16, block, jnp.zeros((16,), jnp.float32))
        out_h[s] = out_cp(k)
    out_h[0].wait()
    out_h[1].wait()


@jax.jit
def kernel(X_in):
    y = jnp.transpose(X_in, (0, 2, 3, 1))
    y = y.reshape(_B, _D, _N // 8, 8, _L // 128, 128)
    y = jnp.transpose(y, (0, 1, 2, 4, 3, 5))
    y = y.reshape(_ROWS, 128)
    run = pl.kernel(
        _scan_body,
        out_type=jax.ShapeDtypeStruct((_ROWS, 128), jnp.float32),
        scratch_types=[
            pltpu.VMEM((_SROWS, 128), jnp.float32),
            pltpu.VMEM((_SROWS, 128), jnp.float32),
            pltpu.SemaphoreType.DMA,
            pltpu.SemaphoreType.DMA,
            pltpu.SemaphoreType.DMA,
            pltpu.SemaphoreType.DMA,
        ],
        mesh=plsc.VectorSubcoreMesh(core_axis_name="c", subcore_axis_name="s"),
        compiler_params=pltpu.CompilerParams(needs_layout_passes=False),
    )
    z = run(y)
    z = z.reshape(_B, _D, _N // 8, _L // 128, 8, 128)
    z = jnp.transpose(z, (0, 1, 2, 4, 3, 5))
    z = z.reshape(_B, _D, _N, _L)
    return jnp.transpose(z, (0, 3, 1, 2))


# speedup vs baseline: 2.2871x; 2.2871x over previous
import jax
import jax.numpy as jnp
from jax import lax
from jax.experimental import pallas as pl
from jax.experimental.pallas import tpu as pltpu
from jax.experimental.pallas import tpu_sc as plsc

_B, _L, _D, _N = 8, 2048, 16, 32
_NC, _NS = 2, 16
_NW = _NC * _NS
_WORDS = _B * _D * _N * _L
_SW = 16 * _L
_UNITS = _WORDS // _SW
_UPW = _UNITS // _NW


def _scan_body(x_hbm, out_hbm, buf0, buf1, is0, is1, os0, os1):
    wid = lax.axis_index("s") * _NC + lax.axis_index("c")
    u0 = wid * _UPW
    bufs = (buf0, buf1)
    isems, osems = (is0, is1), (os0, os1)

    def in_cp(k):
        return pltpu.async_copy(
            x_hbm.at[pl.ds((u0 + k) * _SW, _SW)], bufs[k % 2], isems[k % 2])

    def out_cp(k):
        return pltpu.async_copy(
            bufs[k % 2], out_hbm.at[pl.ds((u0 + k) * _SW, _SW)], osems[k % 2])

    lane = lax.iota(jnp.int32, 16)
    rowbase = (lane // 8) * 16384 + (lane % 8) * 128

    def adj_const(p):
        return jnp.where(lane == 127 - p, 897, 1).astype(jnp.int32)

    in_h = [in_cp(0), None]
    out_h = [None, None]
    for k in range(_UPW):
        s = k % 2
        if k + 1 < _UPW:
            if out_h[(k + 1) % 2] is not None:
                out_h[(k + 1) % 2].wait()
                out_h[(k + 1) % 2] = None
            in_h[(k + 1) % 2] = in_cp(k + 1)
        in_h[s].wait()
        ib = bufs[s]

        acc = jnp.zeros((16,), jnp.float32)
        flat = rowbase - 15 + lane

        for t in range(-15, 0):
            m = lane >= -t
            v = plsc.load_gather(ib, [flat], mask=m)
            acc = acc + jnp.where(m, v, 0.0)
            plsc.store_scatter(ib, [flat], acc, mask=m)
            flat = flat + 1

        def seg_a(acc_flat, n=112):
            @plsc.parallel_loop(0, n, unroll=8, carry=acc_flat)
            def body(t, c):
                a, f = c
                v = plsc.load_gather(ib, [f])
                a = a + v
                plsc.store_scatter(ib, [f], a)
                return (a, f + 1)
            return body

        def seg_b(acc_flat, tail):
            a, f = acc_flat
            for p in range(112, 128):
                if tail and p > 112:
                    m = lane <= 127 - p
                    v = plsc.load_gather(ib, [f], mask=m)
                    a = a + jnp.where(m, v, 0.0)
                    plsc.store_scatter(ib, [f], a, mask=m)
                else:
                    v = plsc.load_gather(ib, [f])
                    a = a + v
                    plsc.store_scatter(ib, [f], a)
                f = f + adj_const(p)
            return (a, f)

        def block(q, acc_flat):
            return seg_b(seg_a(acc_flat), tail=False)

        acc_flat = lax.fori_loop(0, 15, block, (acc, flat))
        seg_b(seg_a(acc_flat), tail=True)

        out_h[s] = out_cp(k)
    out_h[0].wait()
    out_h[1].wait()


@jax.jit
def kernel(X_in):
    y = jnp.transpose(X_in, (0, 2, 3, 1))
    y = y.reshape(_B, _D, _N // 8, 8, _L // 128, 128)
    y = jnp.transpose(y, (0, 1, 2, 4, 3, 5))
    y = y.reshape(_WORDS)
    run = pl.kernel(
        _scan_body,
        out_type=jax.ShapeDtypeStruct((_WORDS,), jnp.float32),
        scratch_types=[
            pltpu.VMEM((_SW,), jnp.float32),
            pltpu.VMEM((_SW,), jnp.float32),
            pltpu.SemaphoreType.DMA,
            pltpu.SemaphoreType.DMA,
            pltpu.SemaphoreType.DMA,
            pltpu.SemaphoreType.DMA,
        ],
        mesh=plsc.VectorSubcoreMesh(core_axis_name="c", subcore_axis_name="s"),
        compiler_params=pltpu.CompilerParams(needs_layout_passes=False),
    )
    z = run(y)
    z = z.reshape(_B, _D, _N // 8, _L // 128, 8, 128)
    z = jnp.transpose(z, (0, 1, 2, 4, 3, 5))
    z = z.reshape(_B, _D, _N, _L)
    return jnp.transpose(z, (0, 3, 1, 2))


# speedup vs baseline: 2.7531x; 1.2038x over previous
import jax
import jax.numpy as jnp
from jax import lax
from jax.experimental import pallas as pl
from jax.experimental.pallas import tpu as pltpu
from jax.experimental.pallas import tpu_sc as plsc

_B, _L, _D, _N = 8, 2048, 16, 32
_NC, _NS = 2, 16
_NW = _NC * _NS
_WORDS = _B * _D * _N * _L
_SW = 16 * _L
_UNITS = _WORDS // _SW
_UPW = _UNITS // _NW


def _scan_body(x_hbm, out_hbm, buf0, buf1, is0, is1, os0, os1):
    wid = lax.axis_index("s") * _NC + lax.axis_index("c")
    u0 = wid * _UPW
    bufs = (buf0, buf1)
    isems, osems = (is0, is1), (os0, os1)

    def in_cp(k):
        return pltpu.async_copy(
            x_hbm.at[pl.ds((u0 + k) * _SW, _SW)], bufs[k % 2], isems[k % 2])

    def out_cp(k):
        return pltpu.async_copy(
            bufs[k % 2], out_hbm.at[pl.ds((u0 + k) * _SW, _SW)], osems[k % 2])

    lane = lax.iota(jnp.int32, 16)
    rowbase = (lane // 8) * 16384 + (lane % 8) * 128

    def adj_const(t):
        return jnp.where(lane == 127 - t, 897, 1).astype(jnp.int32)

    in_h = [in_cp(0), None]
    out_h = [None, None]
    for k in range(_UPW):
        s = k % 2
        if k + 1 < _UPW:
            if out_h[(k + 1) % 2] is not None:
                out_h[(k + 1) % 2].wait()
                out_h[(k + 1) % 2] = None
            in_h[(k + 1) % 2] = in_cp(k + 1)
        in_h[s].wait()
        ib = bufs[s]

        acc = jnp.zeros((16,), jnp.float32)
        flat = rowbase - 15 + lane

        @plsc.parallel_loop(-15, 0, carry=(acc, flat))
        def lead(t, c):
            a, f = c
            m = lane >= -t
            v = plsc.load_gather(ib, [f], mask=m)
            a = a + jnp.where(m, v, 0.0)
            plsc.store_scatter(ib, [f], a, mask=m)
            return (a, f + 1)

        def block(q, acc_flat):
            @plsc.parallel_loop(0, 128, unroll=8, carry=acc_flat)
            def body(t, c):
                a, f = c
                v = plsc.load_gather(ib, [f])
                a = a + v
                plsc.store_scatter(ib, [f], a)
                return (a, f + adj_const(t))
            return body

        acc_flat = lax.fori_loop(0, 15, block, lead)

        @plsc.parallel_loop(0, 128, unroll=8, carry=acc_flat)
        def tail(t, c):
            a, f = c
            m = lane <= 127 - t
            v = plsc.load_gather(ib, [f], mask=m)
            a = a + jnp.where(m, v, 0.0)
            plsc.store_scatter(ib, [f], a, mask=m)
            return (a, f + adj_const(t))
        del tail

        out_h[s] = out_cp(k)
    out_h[0].wait()
    out_h[1].wait()


@jax.jit
def kernel(X_in):
    y = jnp.transpose(X_in, (0, 2, 3, 1))
    y = y.reshape(_B, _D, _N // 8, 8, _L // 128, 128)
    y = jnp.transpose(y, (0, 1, 2, 4, 3, 5))
    y = y.reshape(_WORDS)
    run = pl.kernel(
        _scan_body,
        out_type=jax.ShapeDtypeStruct((_WORDS,), jnp.float32),
        scratch_types=[
            pltpu.VMEM((_SW,), jnp.float32),
            pltpu.VMEM((_SW,), jnp.float32),
            pltpu.SemaphoreType.DMA,
            pltpu.SemaphoreType.DMA,
            pltpu.SemaphoreType.DMA,
            pltpu.SemaphoreType.DMA,
        ],
        mesh=plsc.VectorSubcoreMesh(core_axis_name="c", subcore_axis_name="s"),
        compiler_params=pltpu.CompilerParams(needs_layout_passes=False),
    )
    z = run(y)
    z = z.reshape(_B, _D, _N // 8, _L // 128, 8, 128)
    z = jnp.transpose(z, (0, 1, 2, 4, 3, 5))
    z = z.reshape(_B, _D, _N, _L)
    return jnp.transpose(z, (0, 3, 1, 2))


# speedup vs baseline: 2.9932x; 1.0872x over previous
import jax
import jax.numpy as jnp
from jax import lax
from jax.experimental import pallas as pl
from jax.experimental.pallas import tpu as pltpu
from jax.experimental.pallas import tpu_sc as plsc

_B, _L, _D, _N = 8, 2048, 16, 32
_NC, _NS = 2, 16
_NW = _NC * _NS
_WORDS = _B * _D * _N * _L
_SW = 16 * _L
_UNITS = _WORDS // _SW
_UPW = _UNITS // _NW


def _scan_body(x_hbm, out_hbm, buf0, buf1, is0, is1, os0, os1):
    wid = lax.axis_index("s") * _NC + lax.axis_index("c")
    u0 = wid * _UPW
    bufs = (buf0, buf1)
    isems, osems = (is0, is1), (os0, os1)

    def in_cp(p):
        s = p % 2
        return pltpu.async_copy(
            x_hbm.at[pl.ds((u0 + 2 * p) * _SW, 2 * _SW)],
            bufs[s], isems[s])

    def out_cp(p):
        s = p % 2
        return pltpu.async_copy(
            bufs[s], out_hbm.at[pl.ds((u0 + 2 * p) * _SW, 2 * _SW)],
            osems[s])

    lane = lax.iota(jnp.int32, 16)
    rowbase = (lane // 8) * 16384 + (lane % 8) * 128

    def adj_const(t):
        return jnp.where(lane == 127 - t, 897, 1).astype(jnp.int32)

    npairs = _UPW // 2
    in_h = [in_cp(0), None]
    out_h = [None, None]
    for p in range(npairs):
        s = p % 2
        if p + 1 < npairs:
            if out_h[1 - s] is not None:
                out_h[1 - s].wait()
                out_h[1 - s] = None
            in_h[1 - s] = in_cp(p + 1)
        in_h[s].wait()
        ib0 = bufs[s]

        acc0 = jnp.zeros((16,), jnp.float32)
        acc1 = jnp.zeros((16,), jnp.float32)
        flat = rowbase - 15 + lane

        @plsc.parallel_loop(-15, 0, carry=(acc0, acc1, flat))
        def lead(t, c):
            a0, a1, f = c
            m = lane >= -t
            v0 = plsc.load_gather(ib0, [f], mask=m)
            v1 = plsc.load_gather(ib0, [f + _SW], mask=m)
            a0 = a0 + jnp.where(m, v0, 0.0)
            a1 = a1 + jnp.where(m, v1, 0.0)
            plsc.store_scatter(ib0, [f], a0, mask=m)
            plsc.store_scatter(ib0, [f + _SW], a1, mask=m)
            return (a0, a1, f + 1)

        def block(q, carry):
            @plsc.parallel_loop(0, 128, unroll=8, carry=carry)
            def body(t, c):
                a0, a1, f = c
                v0 = plsc.load_gather(ib0, [f])
                v1 = plsc.load_gather(ib0, [f + _SW])
                a0 = a0 + v0
                a1 = a1 + v1
                plsc.store_scatter(ib0, [f], a0)
                plsc.store_scatter(ib0, [f + _SW], a1)
                return (a0, a1, f + adj_const(t))
            return body

        carry = lax.fori_loop(0, 15, block, lead)

        @plsc.parallel_loop(0, 128, unroll=8, carry=carry)
        def tail(t, c):
            a0, a1, f = c
            m = lane <= 127 - t
            v0 = plsc.load_gather(ib0, [f], mask=m)
            v1 = plsc.load_gather(ib0, [f + _SW], mask=m)
            a0 = a0 + jnp.where(m, v0, 0.0)
            a1 = a1 + jnp.where(m, v1, 0.0)
            plsc.store_scatter(ib0, [f], a0, mask=m)
            plsc.store_scatter(ib0, [f + _SW], a1, mask=m)
            return (a0, a1, f + adj_const(t))
        del tail

        out_h[s] = out_cp(p)
    out_h[0].wait()
    if out_h[1] is not None:
        out_h[1].wait()


@jax.jit
def kernel(X_in):
    y = jnp.transpose(X_in, (0, 2, 3, 1))
    y = y.reshape(_B, _D, _N // 8, 8, _L // 128, 128)
    y = jnp.transpose(y, (0, 1, 2, 4, 3, 5))
    y = y.reshape(_WORDS)
    run = pl.kernel(
        _scan_body,
        out_type=jax.ShapeDtypeStruct((_WORDS,), jnp.float32),
        scratch_types=[
            pltpu.VMEM((2 * _SW,), jnp.float32),
            pltpu.VMEM((2 * _SW,), jnp.float32),
            pltpu.SemaphoreType.DMA,
            pltpu.SemaphoreType.DMA,
            pltpu.SemaphoreType.DMA,
            pltpu.SemaphoreType.DMA,
        ],
        mesh=plsc.VectorSubcoreMesh(core_axis_name="c", subcore_axis_name="s"),
        compiler_params=pltpu.CompilerParams(needs_layout_passes=False),
    )
    z = run(y)
    z = z.reshape(_B, _D, _N // 8, _L // 128, 8, 128)
    z = jnp.transpose(z, (0, 1, 2, 4, 3, 5))
    z = z.reshape(_B, _D, _N, _L)
    return jnp.transpose(z, (0, 3, 1, 2))


# speedup vs baseline: 3.0079x; 1.0049x over previous
import jax
import jax.numpy as jnp
from jax import lax
from jax.experimental import pallas as pl
from jax.experimental.pallas import tpu as pltpu
from jax.experimental.pallas import tpu_sc as plsc

_B, _L, _D, _N = 8, 2048, 16, 32
_NC, _NS = 2, 16
_NW = _NC * _NS
_WORDS = _B * _D * _N * _L
_SW = 16 * _L
_UNITS = _WORDS // _SW
_UPW = _UNITS // _NW


def _scan_body(x_hbm, out_hbm, buf0, buf1, is0, is1, os0, os1):
    wid = lax.axis_index("s") * _NC + lax.axis_index("c")
    u0 = wid * _UPW
    bufs = (buf0, buf1)
    isems, osems = (is0, is1), (os0, os1)

    def in_cp(p):
        s = p % 2
        return pltpu.async_copy(
            x_hbm.at[pl.ds((u0 + 2 * p) * _SW, 2 * _SW)],
            bufs[s], isems[s])

    def out_cp(p):
        s = p % 2
        return pltpu.async_copy(
            bufs[s], out_hbm.at[pl.ds((u0 + 2 * p) * _SW, 2 * _SW)],
            osems[s])

    lane = lax.iota(jnp.int32, 16)
    rowbase = (lane // 8) * 16384 + (lane % 8) * 128


    npairs = _UPW // 2
    in_h = [in_cp(0), None]
    out_h = [None, None]
    for p in range(npairs):
        s = p % 2
        if p + 1 < npairs:
            if out_h[1 - s] is not None:
                out_h[1 - s].wait()
                out_h[1 - s] = None
            in_h[1 - s] = in_cp(p + 1)
        in_h[s].wait()
        ib0 = bufs[s]

        acc0 = jnp.zeros((16,), jnp.float32)
        acc1 = jnp.zeros((16,), jnp.float32)
        flat = rowbase - 15 + lane

        @plsc.parallel_loop(-15, 0, carry=(acc0, acc1, flat))
        def lead(t, c):
            a0, a1, f = c
            m = lane >= -t
            v0 = plsc.load_gather(ib0, [f], mask=m)
            v1 = plsc.load_gather(ib0, [f + _SW], mask=m)
            a0 = a0 + jnp.where(m, v0, 0.0)
            a1 = a1 + jnp.where(m, v1, 0.0)
            plsc.store_scatter(ib0, [f], a0, mask=m)
            plsc.store_scatter(ib0, [f + _SW], a1, mask=m)
            return (a0, a1, f + 1)

        def fast112(carry):
            @plsc.parallel_loop(0, 112, unroll=8, carry=carry)
            def body(t, c):
                a0, a1, f = c
                v0 = plsc.load_gather(ib0, [f])
                v1 = plsc.load_gather(ib0, [f + _SW])
                a0 = a0 + v0
                a1 = a1 + v1
                plsc.store_scatter(ib0, [f], a0)
                plsc.store_scatter(ib0, [f + _SW], a1)
                return (a0, a1, f + 1)
            return body

        def block(q, carry):
            @plsc.parallel_loop(112, 128, unroll=8, carry=fast112(carry))
            def bend(t, c):
                a0, a1, f = c
                v0 = plsc.load_gather(ib0, [f])
                v1 = plsc.load_gather(ib0, [f + _SW])
                a0 = a0 + v0
                a1 = a1 + v1
                plsc.store_scatter(ib0, [f], a0)
                plsc.store_scatter(ib0, [f + _SW], a1)
                return (a0, a1,
                        f + jnp.where(lane == 127 - t, 897, 1).astype(jnp.int32))
            return bend

        carry = lax.fori_loop(0, 15, block, lead)

        @plsc.parallel_loop(112, 128, unroll=8, carry=fast112(carry))
        def tend(t, c):
            a0, a1, f = c
            m = lane <= 127 - t
            v0 = plsc.load_gather(ib0, [f], mask=m)
            v1 = plsc.load_gather(ib0, [f + _SW], mask=m)
            a0 = a0 + jnp.where(m, v0, 0.0)
            a1 = a1 + jnp.where(m, v1, 0.0)
            plsc.store_scatter(ib0, [f], a0, mask=m)
            plsc.store_scatter(ib0, [f + _SW], a1, mask=m)
            return (a0, a1, f + 1)
        del tend

        out_h[s] = out_cp(p)
    out_h[0].wait()
    if out_h[1] is not None:
        out_h[1].wait()


@jax.jit
def kernel(X_in):
    y = jnp.transpose(X_in, (0, 2, 3, 1))
    y = y.reshape(_B, _D, _N // 8, 8, _L // 128, 128)
    y = jnp.transpose(y, (0, 1, 2, 4, 3, 5))
    y = y.reshape(_WORDS)
    run = pl.kernel(
        _scan_body,
        out_type=jax.ShapeDtypeStruct((_WORDS,), jnp.float32),
        scratch_types=[
            pltpu.VMEM((2 * _SW,), jnp.float32),
            pltpu.VMEM((2 * _SW,), jnp.float32),
            pltpu.SemaphoreType.DMA,
            pltpu.SemaphoreType.DMA,
            pltpu.SemaphoreType.DMA,
            pltpu.SemaphoreType.DMA,
        ],
        mesh=plsc.VectorSubcoreMesh(core_axis_name="c", subcore_axis_name="s"),
        compiler_params=pltpu.CompilerParams(needs_layout_passes=False),
    )
    z = run(y)
    z = z.reshape(_B, _D, _N // 8, _L // 128, 8, 128)
    z = jnp.transpose(z, (0, 1, 2, 4, 3, 5))
    z = z.reshape(_B, _D, _N, _L)
    return jnp.transpose(z, (0, 3, 1, 2))
